# Initial kernel scaffold; baseline (speedup 1.0000x reference)
import jax
import jax.numpy as jnp
from jax.experimental import pallas as pl


def kernel(queries, keys, values):
    raise NotImplementedError("write your pallas kernel here")


# speedup vs baseline: 2.1400x; 2.1400x over previous
import math

import numpy as np
import jax
import jax.numpy as jnp
from jax.experimental import pallas as pl
from jax.experimental.pallas import tpu as pltpu

_L = 2048
_H = 12
_E = 64
_FACTOR = 5
_U = _FACTOR * int(np.ceil(np.log(_L)))
_SAMPLE = _FACTOR * int(np.ceil(np.log(_L)))
_SCALE = 1.0 / math.sqrt(_E)
_RB = 256
_KB = 256
_NEG = -1e30


def _build_counts() -> np.ndarray:
    idx = np.asarray(
        jax.random.randint(jax.random.key(42), (_L, _SAMPLE), 0, _L),
        dtype=np.int64,
    )
    c = np.zeros((_L, _L), np.int8)
    np.add.at(c, (np.arange(_L)[:, None], idx), 1)
    return c


_COUNTS = _build_counts()


def _score_kernel(q_ref, k_ref, v_ref, c_ref, mtop_ref, attn_ref, upd_ref, vmean_ref):
    q = q_ref[0]
    k = k_ref[0]
    v = v_ref[0]

    run_max = jnp.full((_L, 1), _NEG, jnp.float32)
    run_sum = jnp.zeros((_L, 1), jnp.float32)
    for kb in range(_L // _KB):
        kcols = k[kb * _KB:(kb + 1) * _KB, :]
        s_blk = jax.lax.dot_general(
            q, kcols, (((1,), (1,)), ((), ())),
            preferred_element_type=jnp.float32)
        c_blk = c_ref[:, kb * _KB:(kb + 1) * _KB].astype(jnp.float32)
        run_max = jnp.maximum(
            run_max,
            jnp.max(jnp.where(c_blk > 0.0, s_blk, _NEG), axis=1, keepdims=True))
        run_sum = run_sum + jnp.sum(s_blk * c_blk, axis=1, keepdims=True)
    m = run_max - run_sum * (1.0 / _L)

    iota_col = jax.lax.broadcasted_iota(jnp.int32, (_L, 1), 0).astype(jnp.float32)
    iota_u = jax.lax.broadcasted_iota(jnp.int32, (1, _U), 1).astype(jnp.float32)
    iota_row = jax.lax.broadcasted_iota(jnp.int32, (1, _L), 1).astype(jnp.float32)
    work = m
    mtop = jnp.zeros((1, _U), jnp.float32)
    oh_rows = []
    for u in range(_U):
        cur = jnp.max(work, axis=(0, 1), keepdims=True)
        idx = jnp.min(jnp.where(work == cur, iota_col, float(_L)),
                      axis=(0, 1), keepdims=True)
        mtop = mtop + idx * (iota_u == float(u)).astype(jnp.float32)
        oh_rows.append((iota_row == idx).astype(jnp.float32))
        work = jnp.where(iota_col == idx, _NEG, work)
    onehot = jnp.concatenate(oh_rows, axis=0)

    q_red = jax.lax.dot_general(
        onehot, q, (((1,), (0,)), ((), ())),
        precision=jax.lax.Precision.HIGHEST,
        preferred_element_type=jnp.float32)
    scores = jax.lax.dot_general(
        q_red, k, (((1,), (1,)), ((), ())),
        preferred_element_type=jnp.float32) * _SCALE
    smax = jnp.max(scores, axis=1, keepdims=True)
    ex = jnp.exp(scores - smax)
    attn = ex / jnp.sum(ex, axis=1, keepdims=True)
    upd = jax.lax.dot_general(
        attn, v, (((1,), (0,)), ((), ())),
        preferred_element_type=jnp.float32)

    mtop_ref[0] = mtop
    attn_ref[0] = attn
    upd_ref[0] = upd
    vmean_ref[0] = jnp.mean(v, axis=0, keepdims=True)


def _write_kernel(mtop_ref, attn_ref, upd_ref, vmean_ref, attns_ref, ctx_ref):
    rb = pl.program_id(1)
    mtop = mtop_ref[0]
    rows = (jax.lax.broadcasted_iota(jnp.int32, (_RB, 1), 0)
            + rb * _RB).astype(jnp.float32)
    oh = (rows == mtop).astype(jnp.float32)
    sel = jnp.sum(oh, axis=1, keepdims=True)
    attn_blk = jax.lax.dot_general(
        oh, attn_ref[0], (((1,), (0,)), ((), ())),
        precision=jax.lax.Precision.HIGHEST,
        preferred_element_type=jnp.float32)
    attns_ref[0] = attn_blk + (1.0 - sel) * (1.0 / _L)
    ctx_blk = jax.lax.dot_general(
        oh, upd_ref[0], (((1,), (0,)), ((), ())),
        precision=jax.lax.Precision.HIGHEST,
        preferred_element_type=jnp.float32)
    ctx_ref[0] = ctx_blk + (1.0 - sel) * vmean_ref[0]


def kernel(queries, keys, values):
    b, l, h, e = queries.shape
    assert (b, l, h, e) == (1, _L, _H, _E)
    qh = jnp.transpose(queries[0], (1, 0, 2))
    kh = jnp.transpose(keys[0], (1, 0, 2))
    vh = jnp.transpose(values[0], (1, 0, 2))
    c = jnp.asarray(_COUNTS)

    head_spec = pl.BlockSpec((1, _L, _E), lambda hh: (hh, 0, 0))
    mtop, attn, upd, vmean = pl.pallas_call(
        _score_kernel,
        grid=(_H,),
        in_specs=[
            head_spec, head_spec, head_spec,
            pl.BlockSpec((_L, _L), lambda hh: (0, 0)),
        ],
        out_specs=[
            pl.BlockSpec((1, 1, _U), lambda hh: (hh, 0, 0)),
            pl.BlockSpec((1, _U, _L), lambda hh: (hh, 0, 0)),
            pl.BlockSpec((1, _U, _E), lambda hh: (hh, 0, 0)),
            pl.BlockSpec((1, 1, _E), lambda hh: (hh, 0, 0)),
        ],
        out_shape=[
            jax.ShapeDtypeStruct((_H, 1, _U), jnp.float32),
            jax.ShapeDtypeStruct((_H, _U, _L), jnp.float32),
            jax.ShapeDtypeStruct((_H, _U, _E), jnp.float32),
            jax.ShapeDtypeStruct((_H, 1, _E), jnp.float32),
        ],
    )(qh, kh, vh, c)

    n_rb = _L // _RB
    attns, ctx = pl.pallas_call(
        _write_kernel,
        grid=(_H, n_rb),
        in_specs=[
            pl.BlockSpec((1, 1, _U), lambda hh, rr: (hh, 0, 0)),
            pl.BlockSpec((1, _U, _L), lambda hh, rr: (hh, 0, 0)),
            pl.BlockSpec((1, _U, _E), lambda hh, rr: (hh, 0, 0)),
            pl.BlockSpec((1, 1, _E), lambda hh, rr: (hh, 0, 0)),
        ],
        out_specs=[
            pl.BlockSpec((1, _RB, _L), lambda hh, rr: (hh, rr, 0)),
            pl.BlockSpec((1, _RB, _E), lambda hh, rr: (hh, rr, 0)),
        ],
        out_shape=[
            jax.ShapeDtypeStruct((_H, _L, _L), jnp.float32),
            jax.ShapeDtypeStruct((_H, _L, _E), jnp.float32),
        ],
    )(mtop, attn, upd, vmean)

    context = jnp.transpose(ctx, (1, 0, 2))[None]
    return context, attns[None]


# speedup vs baseline: 3.9403x; 1.8412x over previous
import math

import numpy as np
import jax
import jax.numpy as jnp
from jax.experimental import pallas as pl
from jax.experimental.pallas import tpu as pltpu

_L = 2048
_H = 12
_E = 64
_FACTOR = 5
_U = _FACTOR * int(np.ceil(np.log(_L)))
_SAMPLE = _FACTOR * int(np.ceil(np.log(_L)))
_SCALE = 1.0 / math.sqrt(_E)
_RB = 256
_QB = 256
_NQ = _L // _QB
_NEG = -1e30


def _threefry2x32(k1, k2, x0, x1):
    k1 = np.uint32(k1)
    k2 = np.uint32(k2)
    ks = [k1, k2, np.uint32(k1 ^ k2 ^ np.uint32(0x1BD11BDA))]
    rot = ([13, 15, 26, 6], [17, 29, 16, 24])
    x0 = (x0 + ks[0]).astype(np.uint32)
    x1 = (x1 + ks[1]).astype(np.uint32)

    def rotl(v, r):
        return ((v << np.uint32(r)) | (v >> np.uint32(32 - r))).astype(np.uint32)

    ks_rot = [ks[1], ks[2], ks[0]]
    for i in range(5):
        for r in rot[i % 2]:
            x0 = (x0 + x1).astype(np.uint32)
            x1 = x0 ^ rotl(x1, r)
        x0 = (x0 + ks_rot[0]).astype(np.uint32)
        x1 = (x1 + ks_rot[1] + np.uint32(i + 1)).astype(np.uint32)
        ks_rot = ks_rot[1:] + ks_rot[:1]
    return x0, x1


def _sample_indices(seed: int) -> np.ndarray:
    k1 = np.uint32(np.int64(seed) >> np.int64(32))
    k2 = np.uint32(np.int64(seed) & np.int64(0xFFFFFFFF))
    b1, b2 = _threefry2x32(k1, k2, np.zeros(2, np.uint32),
                           np.arange(2, dtype=np.uint32))
    n = _L * _SAMPLE
    c1, c2 = _threefry2x32(b1[1], b2[1], np.zeros(n, np.uint32),
                           np.arange(n, dtype=np.uint32))
    return ((c1 ^ c2) % np.uint32(_L)).astype(np.int64).reshape(_L, _SAMPLE)


def _build_counts_t() -> np.ndarray:
    idx = _sample_indices(42)
    c = np.zeros((_L, _L), np.int8)
    np.add.at(c, (np.arange(_L)[:, None], idx), 1)
    return np.ascontiguousarray(c.T)


_COUNTS_T = _build_counts_t()


def _score_kernel(q_ref, k_ref, v_ref, ct_ref, mtop_ref, attn_ref, upd_ref,
                  vmean_ref):
    q = q_ref[0]
    k = k_ref[0]
    v = v_ref[0]

    m_rows = []
    for qb in range(_NQ):
        q_blk = q[qb * _QB:(qb + 1) * _QB, :]
        st_blk = jax.lax.dot_general(
            k, q_blk, (((1,), (1,)), ((), ())),
            preferred_element_type=jnp.float32)
        ct_blk = ct_ref[:, qb * _QB:(qb + 1) * _QB].astype(jnp.float32)
        mx = jnp.max(jnp.where(ct_blk > 0.0, st_blk, _NEG), axis=0,
                     keepdims=True)
        sm = jnp.sum(st_blk * ct_blk, axis=0, keepdims=True)
        m_rows.append(mx - sm * (1.0 / _L))
    m = jnp.concatenate(m_rows, axis=0)

    iota_flat = (jax.lax.broadcasted_iota(jnp.int32, (_NQ, _QB), 0) * _QB
                 + jax.lax.broadcasted_iota(jnp.int32, (_NQ, _QB), 1)
                 ).astype(jnp.float32)
    iota_u = jax.lax.broadcasted_iota(jnp.int32, (1, _U), 1).astype(jnp.float32)
    iota_row = jax.lax.broadcasted_iota(jnp.int32, (1, _L), 1).astype(jnp.float32)
    work = m
    mtop = jnp.zeros((1, _U), jnp.float32)
    oh_rows = []
    for u in range(_U):
        cur = jnp.max(work, axis=(0, 1), keepdims=True)
        idx = jnp.min(jnp.where(work == cur, iota_flat, float(_L)),
                      axis=(0, 1), keepdims=True)
        mtop = mtop + idx * (iota_u == float(u)).astype(jnp.float32)
        oh_rows.append((iota_row == idx).astype(jnp.float32))
        work = jnp.where(iota_flat == idx, _NEG, work)
    onehot = jnp.concatenate(oh_rows, axis=0)

    q_red = jax.lax.dot_general(
        onehot, q, (((1,), (0,)), ((), ())),
        precision=jax.lax.Precision.HIGHEST,
        preferred_element_type=jnp.float32)
    scores = jax.lax.dot_general(
        q_red, k, (((1,), (1,)), ((), ())),
        preferred_element_type=jnp.float32) * _SCALE
    smax = jnp.max(scores, axis=1, keepdims=True)
    ex = jnp.exp(scores - smax)
    attn = ex / jnp.sum(ex, axis=1, keepdims=True)
    upd = jax.lax.dot_general(
        attn, v, (((1,), (0,)), ((), ())),
        preferred_element_type=jnp.float32)

    mtop_ref[0] = mtop.astype(jnp.int32)
    attn_ref[0] = attn
    upd_ref[0] = upd
    vmean_ref[0] = jnp.mean(v, axis=0, keepdims=True)


def _write_kernel(mtop_sref, attn_ref, upd_ref, vmean_ref, attns_ref, ctx_ref):
    hh = pl.program_id(0)
    rb = pl.program_id(1)
    base = rb * _RB
    attns_ref[0] = jnp.full((_RB, _L), 1.0 / _L, jnp.float32)
    ctx_ref[0] = jnp.broadcast_to(vmean_ref[0], (_RB, _E))
    for u in range(_U):
        row = mtop_sref[hh, u]
        local = row - base

        @pl.when(jnp.logical_and(row >= base, row < base + _RB))
        def _copy_row(u=u, local=local):
            attns_ref[0, pl.ds(local, 1), :] = attn_ref[0, pl.ds(u, 1), :]
            ctx_ref[0, pl.ds(local, 1), :] = upd_ref[0, pl.ds(u, 1), :]


def kernel(queries, keys, values):
    b, l, h, e = queries.shape
    assert (b, l, h, e) == (1, _L, _H, _E)
    qh = jnp.transpose(queries[0], (1, 0, 2))
    kh = jnp.transpose(keys[0], (1, 0, 2))
    vh = jnp.transpose(values[0], (1, 0, 2))
    ct = jnp.asarray(_COUNTS_T)

    head_spec = pl.BlockSpec((1, _L, _E), lambda hh: (hh, 0, 0))
    mtop, attn, upd, vmean = pl.pallas_call(
        _score_kernel,
        grid=(_H,),
        in_specs=[
            head_spec, head_spec, head_spec,
            pl.BlockSpec((_L, _L), lambda hh: (0, 0)),
        ],
        out_specs=[
            pl.BlockSpec((1, 1, _U), lambda hh: (hh, 0, 0)),
            pl.BlockSpec((1, _U, _L), lambda hh: (hh, 0, 0)),
            pl.BlockSpec((1, _U, _E), lambda hh: (hh, 0, 0)),
            pl.BlockSpec((1, 1, _E), lambda hh: (hh, 0, 0)),
        ],
        out_shape=[
            jax.ShapeDtypeStruct((_H, 1, _U), jnp.int32),
            jax.ShapeDtypeStruct((_H, _U, _L), jnp.float32),
            jax.ShapeDtypeStruct((_H, _U, _E), jnp.float32),
            jax.ShapeDtypeStruct((_H, 1, _E), jnp.float32),
        ],
    )(qh, kh, vh, ct)

    n_rb = _L // _RB
    grid_spec = pltpu.PrefetchScalarGridSpec(
        num_scalar_prefetch=1,
        grid=(_H, n_rb),
        in_specs=[
            pl.BlockSpec((1, _U, _L), lambda hh, rr, mref: (hh, 0, 0)),
            pl.BlockSpec((1, _U, _E), lambda hh, rr, mref: (hh, 0, 0)),
            pl.BlockSpec((1, 1, _E), lambda hh, rr, mref: (hh, 0, 0)),
        ],
        out_specs=[
            pl.BlockSpec((1, _RB, _L), lambda hh, rr, mref: (hh, rr, 0)),
            pl.BlockSpec((1, _RB, _E), lambda hh, rr, mref: (hh, rr, 0)),
        ],
    )
    attns, ctx = pl.pallas_call(
        _write_kernel,
        grid_spec=grid_spec,
        out_shape=[
            jax.ShapeDtypeStruct((_H, _L, _L), jnp.float32),
            jax.ShapeDtypeStruct((_H, _L, _E), jnp.float32),
        ],
    )(mtop.reshape(_H, _U), attn, upd, vmean)

    context = jnp.transpose(ctx, (1, 0, 2))[None]
    return context, attns[None]
